# baseline (speedup 1.0000x reference)
import functools

import jax
import jax.numpy as jnp
from jax.experimental import pallas as pl
from jax.experimental.pallas import tpu as pltpu

_F32 = jnp.float32
_BF16 = jnp.bfloat16
_DN0 = (((0,), (0,)), ((), ()))
_DN = (((1,), (0,)), ((), ()))


def _layer1_body(
    adj_ref,
    x_ref,
    w_ref,
    b_ref,
    a8_ref,
    hs_ref,
    ddc_ref,
    acc_ref,
    dda_ref,
    dsc_ref,
    *,
    nb,
    bn,
    inv_r,
):
    i = pl.program_id(0)
    rr = adj_ref.shape[0]
    ones = jnp.ones((bn, 1), _BF16)
    for r in range(rr):
        af = (adj_ref[r] != 0).astype(_F32)
        a8_ref[r] = af.astype(jnp.int8)
        abf = af.astype(_BF16)
        s = jnp.sum(af, axis=1, keepdims=True)
        ds = jax.lax.rsqrt(jnp.maximum(s, 1.0))
        dsc_ref[r, pl.ds(i * bn, bn), :] = ds
        xs = (x_ref[...] * ds).astype(_BF16)
        part = jax.lax.dot_general(abf, xs, _DN0, preferred_element_type=_F32)
        dpart = jax.lax.dot_general(abf, ones, _DN0, preferred_element_type=_F32)

        @pl.when(i == 0)
        def _init():
            acc_ref[r] = part
            dda_ref[r] = dpart

        @pl.when(i != 0)
        def _acc():
            acc_ref[r] += part
            dda_ref[r] += dpart

    @pl.when(i == nb - 1)
    def _fin():
        aggs = []
        for r in range(rr):
            ddc = jax.lax.rsqrt(jnp.maximum(dda_ref[r], 1.0))
            ddc_ref[r] = ddc
            aggs.append(acc_ref[r] * ddc)
        aggcat = jnp.concatenate(aggs, axis=1).astype(_BF16)
        acc = jax.lax.dot_general(aggcat, w_ref[...], _DN, preferred_element_type=_F32)
        h1 = jnp.maximum(acc * inv_r + b_ref[...], 0.0)
        for r in range(rr):
            hs_ref[r] = (h1 * dsc_ref[r]).astype(_BF16)


def _layer2_body(
    a8_ref, hs_ref, ddc_ref, w_ref, b_ref, wg_ref, bg_ref, out_ref, *, inv_r, h
):
    aggs = []
    for r in range(a8_ref.shape[0]):
        agg = jax.lax.dot_general(
            a8_ref[r].astype(_BF16), hs_ref[r], _DN0, preferred_element_type=_F32
        )
        aggs.append(agg * ddc_ref[r])
    aggcat = jnp.concatenate(aggs, axis=1).astype(_BF16)
    acc = jax.lax.dot_general(aggcat, w_ref[...], _DN, preferred_element_type=_F32)
    h2 = (acc * inv_r + b_ref[...]).astype(_BF16)
    gates = (
        jax.lax.dot_general(h2, wg_ref[...], _DN, preferred_element_type=_F32)
        + bg_ref[...]
    )
    i_f = gates[:, 0 * h : 1 * h]
    g_f = gates[:, 1 * h : 2 * h]
    o_f = gates[:, 2 * h : 3 * h]
    i_r = gates[:, 3 * h : 4 * h]
    g_r = gates[:, 4 * h : 5 * h]
    o_r = gates[:, 5 * h : 6 * h]
    h_f = jax.nn.sigmoid(o_f) * jnp.tanh(jax.nn.sigmoid(i_f) * jnp.tanh(g_f))
    h_b = jax.nn.sigmoid(o_r) * jnp.tanh(jax.nn.sigmoid(i_r) * jnp.tanh(g_r))
    out_ref[...] = jnp.concatenate([h_f, h_b], axis=1)


def kernel(
    entity_emb,
    rel_adj_matrices,
    W1,
    b1,
    W2,
    b2,
    w_ih_f,
    w_hh_f,
    b_ih_f,
    b_hh_f,
    w_ih_r,
    w_hh_r,
    b_ih_r,
    b_hh_r,
):
    n, in_dim = entity_emb.shape
    rr = rel_adj_matrices.shape[0]
    hid = W1.shape[2]
    out_dim = W2.shape[2]
    h = out_dim // 2
    bn = 256
    bv = 512
    nb = n // bn
    nv = n // bv

    w1 = W1.reshape(rr * in_dim, hid).astype(_BF16)
    b1w = jnp.mean(b1, axis=0).reshape(1, hid)

    a8, hs, ddc = pl.pallas_call(
        functools.partial(_layer1_body, nb=nb, bn=bn, inv_r=1.0 / rr),
        grid=(nb,),
        in_specs=[
            pl.BlockSpec((rr, bn, n), lambda i: (0, i, 0)),
            pl.BlockSpec((bn, in_dim), lambda i: (i, 0)),
            pl.BlockSpec((rr * in_dim, hid), lambda i: (0, 0)),
            pl.BlockSpec((1, hid), lambda i: (0, 0)),
        ],
        out_specs=[
            pl.BlockSpec((rr, bn, n), lambda i: (0, i, 0)),
            pl.BlockSpec((rr, n, hid), lambda i: (0, 0, 0)),
            pl.BlockSpec((rr, n, 1), lambda i: (0, 0, 0)),
        ],
        out_shape=[
            jax.ShapeDtypeStruct((rr, n, n), jnp.int8),
            jax.ShapeDtypeStruct((rr, n, hid), _BF16),
            jax.ShapeDtypeStruct((rr, n, 1), _F32),
        ],
        scratch_shapes=[
            pltpu.VMEM((rr, n, in_dim), _F32),
            pltpu.VMEM((rr, n, 1), _F32),
            pltpu.VMEM((rr, n, 1), _F32),
        ],
    )(rel_adj_matrices, entity_emb, w1, b1w)

    w2 = W2.reshape(rr * hid, out_dim).astype(_BF16)
    b2w = jnp.mean(b2, axis=0).reshape(1, out_dim)
    wg = (
        jnp.concatenate(
            [w_ih_f[0:h], w_ih_f[2 * h :], w_ih_r[0:h], w_ih_r[2 * h :]], axis=0
        )
        .T.astype(_BF16)
    )
    bgf = b_ih_f + b_hh_f
    bgr = b_ih_r + b_hh_r
    bg = jnp.concatenate([bgf[0:h], bgf[2 * h :], bgr[0:h], bgr[2 * h :]]).reshape(1, 6 * h)

    out = pl.pallas_call(
        functools.partial(_layer2_body, inv_r=1.0 / rr, h=h),
        grid=(nv,),
        in_specs=[
            pl.BlockSpec((rr, n, bv), lambda i: (0, 0, i)),
            pl.BlockSpec((rr, n, hid), lambda i: (0, 0, 0)),
            pl.BlockSpec((rr, bv, 1), lambda i: (0, i, 0)),
            pl.BlockSpec((rr * hid, out_dim), lambda i: (0, 0)),
            pl.BlockSpec((1, out_dim), lambda i: (0, 0)),
            pl.BlockSpec((out_dim, 6 * h), lambda i: (0, 0)),
            pl.BlockSpec((1, 6 * h), lambda i: (0, 0)),
        ],
        out_specs=pl.BlockSpec((bv, out_dim), lambda i: (i, 0)),
        out_shape=jax.ShapeDtypeStruct((n, out_dim), _F32),
    )(a8, hs, ddc, w2, b2w, wg, bg)

    return out


# speedup vs baseline: 1.0604x; 1.0604x over previous
import functools

import jax
import jax.numpy as jnp
from jax.experimental import pallas as pl
from jax.experimental.pallas import tpu as pltpu

_F32 = jnp.float32
_BF16 = jnp.bfloat16
_DN0 = (((0,), (0,)), ((), ()))
_DN = (((1,), (0,)), ((), ()))


def _prep_body(adj_ref, x_ref, a8_ref, dsc_ref, ddr_ref, xs_ref, acc_ref, *, nb):
    i = pl.program_id(0)
    r = pl.program_id(1)
    af = (adj_ref[0] != 0).astype(_F32)
    a8_ref[0] = jnp.transpose(af.astype(jnp.int8), (1, 0))
    s = jnp.sum(af, axis=1, keepdims=True)
    ds = jax.lax.rsqrt(jnp.maximum(s, 1.0))
    dsc_ref[0] = ds
    xs_ref[0] = (x_ref[...] * ds).astype(_BF16)

    part = jnp.sum(af, axis=0, keepdims=True)

    @pl.when(i == 0)
    def _init():
        acc_ref[r] = part

    @pl.when(i != 0)
    def _acc():
        acc_ref[r] += part

    @pl.when(i == nb - 1)
    def _fin():
        ddr_ref[0] = jax.lax.rsqrt(jnp.maximum(acc_ref[r], 1.0))


def _agg_cat(a8_ref, xs_ref, ddc_ref):
    aggs = []
    for r in range(a8_ref.shape[0]):
        agg = jax.lax.dot_general(
            a8_ref[r].astype(_BF16), xs_ref[r], _DN, preferred_element_type=_F32
        )
        aggs.append(agg * ddc_ref[r])
    return jnp.concatenate(aggs, axis=1).astype(_BF16)


def _layers_body(
    a8_ref,
    xs_ref,
    ddc_ref,
    dsc_ref,
    w1_ref,
    b1_ref,
    w2_ref,
    b2_ref,
    wg_ref,
    bg_ref,
    out_ref,
    hs_ref,
    *,
    inv_r,
    h,
    bv,
):
    p = pl.program_id(0)
    i = pl.program_id(1)

    @pl.when(p == 0)
    def _layer1():
        aggcat = _agg_cat(a8_ref, xs_ref, ddc_ref)
        acc = jax.lax.dot_general(aggcat, w1_ref[...], _DN, preferred_element_type=_F32)
        h1 = jnp.maximum(acc * inv_r + b1_ref[...], 0.0)
        for r in range(dsc_ref.shape[0]):
            hs_ref[r, pl.ds(i * bv, bv), :] = (h1 * dsc_ref[r]).astype(_BF16)

    @pl.when(p == 1)
    def _layer2():
        aggcat = _agg_cat(a8_ref, hs_ref, ddc_ref)
        acc = jax.lax.dot_general(aggcat, w2_ref[...], _DN, preferred_element_type=_F32)
        h2 = (acc * inv_r + b2_ref[...]).astype(_BF16)
        gates = (
            jax.lax.dot_general(h2, wg_ref[...], _DN, preferred_element_type=_F32)
            + bg_ref[...]
        )
        i_f = gates[:, 0 * h : 1 * h]
        g_f = gates[:, 1 * h : 2 * h]
        o_f = gates[:, 2 * h : 3 * h]
        i_r = gates[:, 3 * h : 4 * h]
        g_r = gates[:, 4 * h : 5 * h]
        o_r = gates[:, 5 * h : 6 * h]
        h_f = jax.nn.sigmoid(o_f) * jnp.tanh(jax.nn.sigmoid(i_f) * jnp.tanh(g_f))
        h_b = jax.nn.sigmoid(o_r) * jnp.tanh(jax.nn.sigmoid(i_r) * jnp.tanh(g_r))
        out_ref[...] = jnp.concatenate([h_f, h_b], axis=1)


def kernel(
    entity_emb,
    rel_adj_matrices,
    W1,
    b1,
    W2,
    b2,
    w_ih_f,
    w_hh_f,
    b_ih_f,
    b_hh_f,
    w_ih_r,
    w_hh_r,
    b_ih_r,
    b_hh_r,
):
    n, in_dim = entity_emb.shape
    rr = rel_adj_matrices.shape[0]
    hid = W1.shape[2]
    out_dim = W2.shape[2]
    h = out_dim // 2
    bn = 512
    bv = 1024
    nb = n // bn
    nv = n // bv

    a8, dsc, ddr, xs = pl.pallas_call(
        functools.partial(_prep_body, nb=nb),
        grid=(nb, rr),
        in_specs=[
            pl.BlockSpec((1, bn, n), lambda i, r: (r, i, 0)),
            pl.BlockSpec((bn, in_dim), lambda i, r: (i, 0)),
        ],
        out_specs=[
            pl.BlockSpec((1, n, bn), lambda i, r: (r, 0, i)),
            pl.BlockSpec((1, bn, 1), lambda i, r: (r, i, 0)),
            pl.BlockSpec((1, 1, n), lambda i, r: (r, 0, 0)),
            pl.BlockSpec((1, bn, in_dim), lambda i, r: (r, i, 0)),
        ],
        out_shape=[
            jax.ShapeDtypeStruct((rr, n, n), jnp.int8),
            jax.ShapeDtypeStruct((rr, n, 1), _F32),
            jax.ShapeDtypeStruct((rr, 1, n), _F32),
            jax.ShapeDtypeStruct((rr, n, in_dim), _BF16),
        ],
        scratch_shapes=[pltpu.VMEM((rr, 1, n), _F32)],
    )(rel_adj_matrices, entity_emb)

    ddc = jnp.transpose(ddr, (0, 2, 1))
    w1 = W1.reshape(rr * in_dim, hid).astype(_BF16)
    b1w = jnp.mean(b1, axis=0).reshape(1, hid)
    w2 = W2.reshape(rr * hid, out_dim).astype(_BF16)
    b2w = jnp.mean(b2, axis=0).reshape(1, out_dim)
    wg = (
        jnp.concatenate(
            [w_ih_f[0:h], w_ih_f[2 * h :], w_ih_r[0:h], w_ih_r[2 * h :]], axis=0
        )
        .T.astype(_BF16)
    )
    bgf = b_ih_f + b_hh_f
    bgr = b_ih_r + b_hh_r
    bg = jnp.concatenate([bgf[0:h], bgf[2 * h :], bgr[0:h], bgr[2 * h :]]).reshape(1, 6 * h)

    out = pl.pallas_call(
        functools.partial(_layers_body, inv_r=1.0 / rr, h=h, bv=bv),
        grid=(2, nv),
        in_specs=[
            pl.BlockSpec((rr, bv, n), lambda p, i: (0, i, 0)),
            pl.BlockSpec((rr, n, in_dim), lambda p, i: (0, 0, 0)),
            pl.BlockSpec((rr, bv, 1), lambda p, i: (0, i, 0)),
            pl.BlockSpec((rr, bv, 1), lambda p, i: (0, i, 0)),
            pl.BlockSpec((rr * in_dim, hid), lambda p, i: (0, 0)),
            pl.BlockSpec((1, hid), lambda p, i: (0, 0)),
            pl.BlockSpec((rr * hid, out_dim), lambda p, i: (0, 0)),
            pl.BlockSpec((1, out_dim), lambda p, i: (0, 0)),
            pl.BlockSpec((out_dim, 6 * h), lambda p, i: (0, 0)),
            pl.BlockSpec((1, 6 * h), lambda p, i: (0, 0)),
        ],
        out_specs=pl.BlockSpec((bv, out_dim), lambda p, i: (p * i, 0)),
        out_shape=jax.ShapeDtypeStruct((n, out_dim), _F32),
        scratch_shapes=[pltpu.VMEM((rr, n, hid), _BF16)],
    )(a8, xs, ddc, dsc, w1, b1w, w2, b2w, wg, bg)

    return out


# speedup vs baseline: 1.0609x; 1.0005x over previous
import functools

import jax
import jax.numpy as jnp
from jax.experimental import pallas as pl
from jax.experimental.pallas import tpu as pltpu

_F32 = jnp.float32
_BF16 = jnp.bfloat16
_DN0 = (((0,), (0,)), ((), ()))
_DN = (((1,), (0,)), ((), ()))


def _prep_body(adj_ref, x_ref, a8_ref, dsc_ref, ddr_ref, xs_ref, acc_ref, *, nb):
    i = pl.program_id(0)
    r = pl.program_id(1)
    af = (adj_ref[0] != 0).astype(_F32)
    a8_ref[0] = jnp.transpose(af.astype(jnp.int8), (1, 0))
    s = jnp.sum(af, axis=1, keepdims=True)
    ds = jax.lax.rsqrt(jnp.maximum(s, 1.0))
    dsc_ref[0] = ds
    xs_ref[0] = (x_ref[...] * ds).astype(_BF16)

    part = jnp.sum(af, axis=0, keepdims=True)

    @pl.when(i == 0)
    def _init():
        acc_ref[r] = part

    @pl.when(i != 0)
    def _acc():
        acc_ref[r] += part

    @pl.when(i == nb - 1)
    def _fin():
        ddr_ref[0] = jax.lax.rsqrt(jnp.maximum(acc_ref[r], 1.0))


def _agg_cat(a8_ref, xs_ref, ddc_ref):
    aggs = []
    for r in range(a8_ref.shape[0]):
        agg = jax.lax.dot_general(
            a8_ref[r].astype(_BF16), xs_ref[r], _DN, preferred_element_type=_F32
        )
        aggs.append(agg * ddc_ref[r])
    return jnp.concatenate(aggs, axis=1).astype(_BF16)


def _layers_body(
    a8_ref,
    xs_ref,
    ddc_ref,
    dsc_ref,
    w1_ref,
    b1_ref,
    w2_ref,
    b2_ref,
    wg_ref,
    bg_ref,
    out_ref,
    hs_ref,
    *,
    inv_r,
    h,
    bv,
):
    p = pl.program_id(0)
    i = pl.program_id(1)

    @pl.when(p == 0)
    def _layer1():
        aggcat = _agg_cat(a8_ref, xs_ref, ddc_ref)
        acc = jax.lax.dot_general(aggcat, w1_ref[...], _DN, preferred_element_type=_F32)
        h1 = jnp.maximum(acc * inv_r + b1_ref[...], 0.0)
        for r in range(dsc_ref.shape[0]):
            hs_ref[r, pl.ds(i * bv, bv), :] = (h1 * dsc_ref[r]).astype(_BF16)

    @pl.when(p == 1)
    def _layer2():
        aggcat = _agg_cat(a8_ref, hs_ref, ddc_ref)
        acc = jax.lax.dot_general(aggcat, w2_ref[...], _DN, preferred_element_type=_F32)
        h2 = (acc * inv_r + b2_ref[...]).astype(_BF16)
        gates = (
            jax.lax.dot_general(h2, wg_ref[...], _DN, preferred_element_type=_F32)
            + bg_ref[...]
        )
        i_f = gates[:, 0 * h : 1 * h]
        g_f = gates[:, 1 * h : 2 * h]
        o_f = gates[:, 2 * h : 3 * h]
        i_r = gates[:, 3 * h : 4 * h]
        g_r = gates[:, 4 * h : 5 * h]
        o_r = gates[:, 5 * h : 6 * h]
        h_f = jax.nn.sigmoid(o_f) * jnp.tanh(jax.nn.sigmoid(i_f) * jnp.tanh(g_f))
        h_b = jax.nn.sigmoid(o_r) * jnp.tanh(jax.nn.sigmoid(i_r) * jnp.tanh(g_r))
        out_ref[...] = jnp.concatenate([h_f, h_b], axis=1)


def kernel(
    entity_emb,
    rel_adj_matrices,
    W1,
    b1,
    W2,
    b2,
    w_ih_f,
    w_hh_f,
    b_ih_f,
    b_hh_f,
    w_ih_r,
    w_hh_r,
    b_ih_r,
    b_hh_r,
):
    n, in_dim = entity_emb.shape
    rr = rel_adj_matrices.shape[0]
    hid = W1.shape[2]
    out_dim = W2.shape[2]
    h = out_dim // 2
    bn = 512
    bv = 512
    nb = n // bn
    nv = n // bv

    a8, dsc, ddr, xs = pl.pallas_call(
        functools.partial(_prep_body, nb=nb),
        grid=(nb, rr),
        in_specs=[
            pl.BlockSpec((1, bn, n), lambda i, r: (r, i, 0)),
            pl.BlockSpec((bn, in_dim), lambda i, r: (i, 0)),
        ],
        out_specs=[
            pl.BlockSpec((1, n, bn), lambda i, r: (r, 0, i)),
            pl.BlockSpec((1, bn, 1), lambda i, r: (r, i, 0)),
            pl.BlockSpec((1, 1, n), lambda i, r: (r, 0, 0)),
            pl.BlockSpec((1, bn, in_dim), lambda i, r: (r, i, 0)),
        ],
        out_shape=[
            jax.ShapeDtypeStruct((rr, n, n), jnp.int8),
            jax.ShapeDtypeStruct((rr, n, 1), _F32),
            jax.ShapeDtypeStruct((rr, 1, n), _F32),
            jax.ShapeDtypeStruct((rr, n, in_dim), _BF16),
        ],
        scratch_shapes=[pltpu.VMEM((rr, 1, n), _F32)],
    )(rel_adj_matrices, entity_emb)

    ddc = jnp.transpose(ddr, (0, 2, 1))
    w1 = W1.reshape(rr * in_dim, hid).astype(_BF16)
    b1w = jnp.mean(b1, axis=0).reshape(1, hid)
    w2 = W2.reshape(rr * hid, out_dim).astype(_BF16)
    b2w = jnp.mean(b2, axis=0).reshape(1, out_dim)
    wg = (
        jnp.concatenate(
            [w_ih_f[0:h], w_ih_f[2 * h :], w_ih_r[0:h], w_ih_r[2 * h :]], axis=0
        )
        .T.astype(_BF16)
    )
    bgf = b_ih_f + b_hh_f
    bgr = b_ih_r + b_hh_r
    bg = jnp.concatenate([bgf[0:h], bgf[2 * h :], bgr[0:h], bgr[2 * h :]]).reshape(1, 6 * h)

    out = pl.pallas_call(
        functools.partial(_layers_body, inv_r=1.0 / rr, h=h, bv=bv),
        grid=(2, nv),
        in_specs=[
            pl.BlockSpec((rr, bv, n), lambda p, i: (0, i, 0)),
            pl.BlockSpec((rr, n, in_dim), lambda p, i: (0, 0, 0)),
            pl.BlockSpec((rr, bv, 1), lambda p, i: (0, i, 0)),
            pl.BlockSpec((rr, bv, 1), lambda p, i: (0, i, 0)),
            pl.BlockSpec((rr * in_dim, hid), lambda p, i: (0, 0)),
            pl.BlockSpec((1, hid), lambda p, i: (0, 0)),
            pl.BlockSpec((rr * hid, out_dim), lambda p, i: (0, 0)),
            pl.BlockSpec((1, out_dim), lambda p, i: (0, 0)),
            pl.BlockSpec((out_dim, 6 * h), lambda p, i: (0, 0)),
            pl.BlockSpec((1, 6 * h), lambda p, i: (0, 0)),
        ],
        out_specs=pl.BlockSpec((bv, out_dim), lambda p, i: (p * i, 0)),
        out_shape=jax.ShapeDtypeStruct((n, out_dim), _F32),
        scratch_shapes=[pltpu.VMEM((rr, n, hid), _BF16)],
    )(a8, xs, ddc, dsc, w1, b1w, w2, b2w, wg, bg)

    return out
